# baseline (speedup 1.0000x reference)
import functools

import jax
import jax.numpy as jnp
from jax import lax
from jax.experimental import pallas as pl
from jax.experimental.pallas import tpu as pltpu
from jax.experimental.pallas import tpu_sc as plsc

_NC = 2
_NS = 16
_CHUNK = 128


_NSEC = 10


def _make_sc_agg(N, D, E):
    NW = _NC * _NS
    sec_quant = 8 * _NSEC * _CHUNK
    ept = ((-(-E // NW) + sec_quant - 1) // sec_quant) * sec_quant
    n_chunks = ept // _CHUNK
    cps = n_chunks // _NSEC
    e_pad = ept * NW
    n_acc = -(-(N + 1) // (_NS * 8)) * (_NS * 8)
    rpt = n_acc // _NS

    mesh = plsc.VectorSubcoreMesh(core_axis_name="c", subcore_axis_name="s")

    @functools.partial(
        pl.kernel,
        out_type=jax.ShapeDtypeStruct((_NC, n_acc, D), jnp.float32),
        mesh=mesh,
        scratch_types=[
            pltpu.VMEM((n_chunks + 1, _CHUNK), jnp.int32),
            pltpu.VMEM((cps, _CHUNK), jnp.int32),
            pltpu.VMEM((cps, _CHUNK), jnp.int32),
            pltpu.VMEM((_CHUNK, D), jnp.float32),
            pltpu.VMEM((_CHUNK, D), jnp.float32),
            pltpu.VMEM_SHARED((n_acc, D), jnp.float32),
            pltpu.SemaphoreType.DMA,
            pltpu.SemaphoreType.DMA,
            pltpu.SemaphoreType.DMA,
            pltpu.SemaphoreType.DMA,
            pltpu.SemaphoreType.DMA,
        ],
    )
    def agg(x_hbm, row_hbm, col_hbm, zero_hbm, out_hbm, col_v, row_a, row_b,
            buf0, buf1, acc, sem0, sem1, sem_ra, sem_rb, semi):
        c = lax.axis_index("c")
        s = lax.axis_index("s")
        w = c * _NS + s
        cp_c = pltpu.async_copy(col_hbm.at[w], col_v, semi)
        pltpu.async_copy(row_hbm.at[w, pl.ds(0, cps)], row_a, sem_ra)
        pltpu.async_copy(row_hbm.at[w, pl.ds(cps, cps)], row_b, sem_rb)
        pltpu.sync_copy(zero_hbm, acc.at[pl.ds(s * rpt, rpt)])
        cp_c.wait()
        plsc.subcore_barrier()

        pltpu.async_copy(x_hbm.at[col_v.at[0]], buf0, sem0)

        for sec in range(_NSEC):
            rbuf = row_a if sec % 2 == 0 else row_b
            rsem = sem_ra if sec % 2 == 0 else sem_rb
            pltpu.make_async_copy(
                row_hbm.at[w, pl.ds(sec * cps, cps)], rbuf, rsem).wait()

            @pl.loop(0, cps // 2)
            def _pair(p, sec=sec, rbuf=rbuf):
                k = sec * cps + 2 * p
                g1 = pltpu.async_copy(x_hbm.at[col_v.at[k + 1]], buf1, sem1)
                pltpu.make_async_copy(
                    x_hbm.at[col_v.at[k]], buf0, sem0).wait()
                pltpu.sync_copy(buf0, acc.at[rbuf.at[2 * p]], add=True)
                pltpu.async_copy(x_hbm.at[col_v.at[k + 2]], buf0, sem0)
                g1.wait()
                pltpu.sync_copy(buf1, acc.at[rbuf.at[2 * p + 1]], add=True)

            if sec + 2 < _NSEC:
                pltpu.async_copy(
                    row_hbm.at[w, pl.ds((sec + 2) * cps, cps)], rbuf, rsem)

        pltpu.make_async_copy(x_hbm.at[col_v.at[n_chunks]], buf0, sem0).wait()
        plsc.subcore_barrier()
        pltpu.sync_copy(acc.at[pl.ds(s * rpt, rpt)],
                        out_hbm.at[c, pl.ds(s * rpt, rpt)])

    return agg, n_chunks, e_pad, n_acc


def _tc_linear_body(x_ref, p0_ref, p1_ref, w_ref, b_ref, o_ref):
    s = x_ref[...] + p0_ref[...] + p1_ref[...]
    h = lax.dot_general(s, w_ref[...], (((1,), (1,)), ((), ())),
                        preferred_element_type=jnp.float32)
    h = h + b_ref[...]
    o_ref[...] = h * jax.nn.sigmoid(h)


def kernel(x, edge_index, edge_attr, W, b):
    N, D = x.shape
    E = edge_index.shape[1]
    NW = _NC * _NS
    ei = edge_index.astype(jnp.int32)
    row, col = ei[0], ei[1]

    agg_fn, n_chunks, e_pad, n_acc = _make_sc_agg(N, D, E)
    pad = e_pad - E
    row_p = jnp.concatenate([row, jnp.full((pad,), N, jnp.int32)])
    row_p = row_p.reshape(NW, n_chunks, _CHUNK)
    col_p = jnp.concatenate([col, jnp.zeros((pad,), jnp.int32)])
    col_p = col_p.reshape(NW, n_chunks, _CHUNK)
    col_p = jnp.concatenate(
        [col_p, jnp.zeros((NW, 1, _CHUNK), jnp.int32)], axis=1)
    zeros = jnp.zeros((n_acc // _NS, D), jnp.float32)

    parts = agg_fn(x, row_p, col_p, zeros)
    p0 = parts[0, :N]
    p1 = parts[1, :N]

    RB = 1000
    return pl.pallas_call(
        _tc_linear_body,
        grid=(N // RB,),
        in_specs=[
            pl.BlockSpec((RB, D), lambda i: (i, 0)),
            pl.BlockSpec((RB, D), lambda i: (i, 0)),
            pl.BlockSpec((RB, D), lambda i: (i, 0)),
            pl.BlockSpec((D, D), lambda i: (0, 0)),
            pl.BlockSpec((1, D), lambda i: (0, 0)),
        ],
        out_specs=pl.BlockSpec((RB, D), lambda i: (i, 0)),
        out_shape=jax.ShapeDtypeStruct((N, D), jnp.float32),
    )(x, p0, p1, W, b.reshape(1, D))


# speedup vs baseline: 2.5960x; 2.5960x over previous
import functools

import jax
import jax.numpy as jnp
from jax import lax
from jax.experimental import pallas as pl
from jax.experimental.pallas import tpu as pltpu
from jax.experimental.pallas import tpu_sc as plsc

_NC = 2
_NS = 16
_CHUNK = 128


def _make_sc_agg(N, D, E):
    NW = _NC * _NS
    ept = ((-(-E // NW) + _CHUNK - 1) // _CHUNK) * _CHUNK
    n_chunks = ept // _CHUNK
    e_pad = ept * NW
    n_acc = -(-(N + 1) // (_NS * 8)) * (_NS * 8)
    rpt = n_acc // _NS

    mesh = plsc.VectorSubcoreMesh(core_axis_name="c", subcore_axis_name="s")

    @functools.partial(
        pl.kernel,
        out_type=jax.ShapeDtypeStruct((_NC, n_acc, D), jnp.float32),
        mesh=mesh,
        scratch_types=[
            pltpu.VMEM((n_chunks, _CHUNK), jnp.int32),
            pltpu.VMEM((n_chunks, _CHUNK), jnp.int32),
            pltpu.VMEM((_CHUNK, D), jnp.float32),
            pltpu.VMEM_SHARED((n_acc, D), jnp.float32),
            pltpu.SemaphoreType.DMA,
            pltpu.SemaphoreType.DMA,
        ],
    )
    def agg(x_hbm, row_hbm, col_hbm, zero_hbm, out_hbm, col_v, row_v,
            buf0, acc, sem0, semi):
        c = lax.axis_index("c")
        s = lax.axis_index("s")
        w = c * _NS + s
        cp_c = pltpu.async_copy(col_hbm.at[w], col_v, semi)
        cp_r = pltpu.async_copy(row_hbm.at[w], row_v, semi)
        pltpu.sync_copy(zero_hbm, acc.at[pl.ds(s * rpt, rpt)])
        cp_c.wait()
        cp_r.wait()
        plsc.subcore_barrier()

        @pl.loop(0, n_chunks)
        def _chunk(k):
            pltpu.async_copy(x_hbm.at[col_v.at[k]], buf0, sem0).wait()
            pltpu.sync_copy(buf0, acc.at[row_v.at[k]], add=True)

        plsc.subcore_barrier()
        pltpu.sync_copy(acc.at[pl.ds(s * rpt, rpt)],
                        out_hbm.at[c, pl.ds(s * rpt, rpt)])

    return agg, n_chunks, e_pad, n_acc


def _tc_linear_body(x_ref, p0_ref, p1_ref, w_ref, b_ref, o_ref):
    s = x_ref[...] + p0_ref[...] + p1_ref[...]
    h = lax.dot_general(s, w_ref[...], (((1,), (1,)), ((), ())),
                        preferred_element_type=jnp.float32)
    h = h + b_ref[...]
    o_ref[...] = h * jax.nn.sigmoid(h)


def kernel(x, edge_index, edge_attr, W, b):
    N, D = x.shape
    E = edge_index.shape[1]
    NW = _NC * _NS
    ei = edge_index.astype(jnp.int32)
    row, col = ei[0], ei[1]

    agg_fn, n_chunks, e_pad, n_acc = _make_sc_agg(N, D, E)
    pad = e_pad - E
    n_dummy = n_acc - N
    row_p = jnp.concatenate(
        [row, N + (jnp.arange(pad, dtype=jnp.int32) % n_dummy)])
    row_p = row_p.reshape(NW, n_chunks, _CHUNK)
    col_p = jnp.concatenate(
        [col, jnp.arange(pad, dtype=jnp.int32) % N])
    col_p = col_p.reshape(NW, n_chunks, _CHUNK)
    zeros = jnp.zeros((n_acc // _NS, D), jnp.float32)

    parts = agg_fn(x, row_p, col_p, zeros)
    p0 = parts[0, :N]
    p1 = parts[1, :N]

    RB = 1000
    return pl.pallas_call(
        _tc_linear_body,
        grid=(N // RB,),
        in_specs=[
            pl.BlockSpec((RB, D), lambda i: (i, 0)),
            pl.BlockSpec((RB, D), lambda i: (i, 0)),
            pl.BlockSpec((RB, D), lambda i: (i, 0)),
            pl.BlockSpec((D, D), lambda i: (0, 0)),
            pl.BlockSpec((1, D), lambda i: (0, 0)),
        ],
        out_specs=pl.BlockSpec((RB, D), lambda i: (i, 0)),
        out_shape=jax.ShapeDtypeStruct((N, D), jnp.float32),
    )(x, p0, p1, W, b.reshape(1, D))


# speedup vs baseline: 3.2172x; 1.2393x over previous
import functools

import jax
import jax.numpy as jnp
from jax import lax
from jax.experimental import pallas as pl
from jax.experimental.pallas import tpu as pltpu
from jax.experimental.pallas import tpu_sc as plsc

_NC = 2
_NS = 16
_CHUNK = 128
_G = 2


def _make_sc_agg(N, D, E):
    NW = _NC * _NS
    grp = 2 * _G * _CHUNK
    ept = ((-(-E // NW) + grp - 1) // grp) * grp
    n_chunks = ept // _CHUNK
    n_half = n_chunks // 2
    gph = n_half // _G
    e_pad = ept * NW
    n_acc = -(-(N + 1) // (_NS * 8)) * (_NS * 8)
    rpt = n_acc // _NS

    mesh = plsc.VectorSubcoreMesh(core_axis_name="c", subcore_axis_name="s")

    @functools.partial(
        pl.kernel,
        out_type=jax.ShapeDtypeStruct((_NC, n_acc, D), jnp.float32),
        mesh=mesh,
        scratch_types=[
            pltpu.VMEM((n_half, _CHUNK), jnp.int32),
            pltpu.VMEM((n_half, _CHUNK), jnp.int32),
            pltpu.VMEM((_G, _CHUNK, D), jnp.float32),
            pltpu.VMEM_SHARED((n_acc, D), jnp.float32),
            pltpu.SemaphoreType.DMA,
            pltpu.SemaphoreType.DMA,
        ],
    )
    def agg(x_hbm, row_hbm, col_hbm, zero_hbm, out_hbm, col_v, row_v,
            buf0, acc, sem0, semi):
        c = lax.axis_index("c")
        s = lax.axis_index("s")
        w = c * _NS + s
        cp_c = pltpu.async_copy(col_hbm.at[w, pl.ds(0, n_half)], col_v, semi)
        cp_r = pltpu.async_copy(row_hbm.at[w, pl.ds(0, n_half)], row_v, semi)
        pltpu.sync_copy(zero_hbm, acc.at[pl.ds(s * rpt, rpt)])
        cp_c.wait()
        cp_r.wait()
        plsc.subcore_barrier()

        for h in range(2):
            if h == 1:
                cp_c1 = pltpu.async_copy(
                    col_hbm.at[w, pl.ds(n_half, n_half)], col_v, semi)
                cp_r1 = pltpu.async_copy(
                    row_hbm.at[w, pl.ds(n_half, n_half)], row_v, semi)
                cp_c1.wait()
                cp_r1.wait()

            @pl.loop(0, gph)
            def _grp(g):
                k0 = g * _G
                cps = [
                    pltpu.async_copy(
                        x_hbm.at[col_v.at[k0 + i]], buf0.at[i], sem0)
                    for i in range(_G)
                ]
                for i in range(_G):
                    cps[i].wait()
                    pltpu.sync_copy(
                        buf0.at[i], acc.at[row_v.at[k0 + i]], add=True)

        plsc.subcore_barrier()
        pltpu.sync_copy(acc.at[pl.ds(s * rpt, rpt)],
                        out_hbm.at[c, pl.ds(s * rpt, rpt)])

    return agg, n_chunks, e_pad, n_acc


def _tc_linear_body(x_ref, p0_ref, p1_ref, w_ref, b_ref, o_ref):
    s = x_ref[...] + p0_ref[...] + p1_ref[...]
    h = lax.dot_general(s, w_ref[...], (((1,), (1,)), ((), ())),
                        preferred_element_type=jnp.float32)
    h = h + b_ref[...]
    o_ref[...] = h * jax.nn.sigmoid(h)


def kernel(x, edge_index, edge_attr, W, b):
    N, D = x.shape
    E = edge_index.shape[1]
    NW = _NC * _NS
    ei = edge_index.astype(jnp.int32)
    row, col = ei[0], ei[1]

    agg_fn, n_chunks, e_pad, n_acc = _make_sc_agg(N, D, E)
    pad = e_pad - E
    n_dummy = n_acc - N
    row_p = jnp.concatenate(
        [row, N + (jnp.arange(pad, dtype=jnp.int32) % n_dummy)])
    row_p = row_p.reshape(NW, n_chunks, _CHUNK)
    col_p = jnp.concatenate(
        [col, jnp.arange(pad, dtype=jnp.int32) % N])
    col_p = col_p.reshape(NW, n_chunks, _CHUNK)
    zeros = jnp.zeros((n_acc // _NS, D), jnp.float32)

    parts = agg_fn(x, row_p, col_p, zeros)
    p0 = parts[0, :N]
    p1 = parts[1, :N]

    RB = 1000
    return pl.pallas_call(
        _tc_linear_body,
        grid=(N // RB,),
        in_specs=[
            pl.BlockSpec((RB, D), lambda i: (i, 0)),
            pl.BlockSpec((RB, D), lambda i: (i, 0)),
            pl.BlockSpec((RB, D), lambda i: (i, 0)),
            pl.BlockSpec((D, D), lambda i: (0, 0)),
            pl.BlockSpec((1, D), lambda i: (0, 0)),
        ],
        out_specs=pl.BlockSpec((RB, D), lambda i: (i, 0)),
        out_shape=jax.ShapeDtypeStruct((N, D), jnp.float32),
    )(x, p0, p1, W, b.reshape(1, D))


# speedup vs baseline: 3.7024x; 1.1508x over previous
import functools

import jax
import jax.numpy as jnp
from jax import lax
from jax.experimental import pallas as pl
from jax.experimental.pallas import tpu as pltpu
from jax.experimental.pallas import tpu_sc as plsc

_NC = 2
_NS = 16
_CHUNK = 128
_G = 2


def _make_sc_agg(N, D, E):
    NW = _NC * _NS
    grp = 2 * _G * _CHUNK
    ept = ((-(-E // NW) + grp - 1) // grp) * grp
    n_chunks = ept // _CHUNK
    n_half = n_chunks // 2
    gph = n_half // _G
    e_pad = ept * NW
    n_acc = -(-(N + 1) // (_NS * 8)) * (_NS * 8)
    rpt = n_acc // _NS

    mesh = plsc.VectorSubcoreMesh(core_axis_name="c", subcore_axis_name="s")

    @functools.partial(
        pl.kernel,
        out_type=jax.ShapeDtypeStruct((_NC, n_acc, D), jnp.float32),
        mesh=mesh,
        scratch_types=[
            pltpu.VMEM((n_half, _CHUNK), jnp.int32),
            pltpu.VMEM((n_half, _CHUNK), jnp.int32),
            pltpu.VMEM((_G, _CHUNK, D), jnp.float32),
            pltpu.VMEM_SHARED((n_acc, D), jnp.float32),
            pltpu.SemaphoreType.DMA,
            pltpu.SemaphoreType.DMA,
        ],
    )
    def agg(x_hbm, row_hbm, col_hbm, zero_hbm, out_hbm, col_v, row_v,
            buf0, acc, sem0, semi):
        c = lax.axis_index("c")
        s = lax.axis_index("s")
        w = c * _NS + s
        cp_c = pltpu.async_copy(col_hbm.at[w, pl.ds(0, n_half)], col_v, semi)
        cp_r = pltpu.async_copy(row_hbm.at[w, pl.ds(0, n_half)], row_v, semi)
        pltpu.sync_copy(zero_hbm, acc.at[pl.ds(s * rpt, rpt)])
        cp_c.wait()
        cp_r.wait()
        plsc.subcore_barrier()

        for h in range(2):
            if h == 1:
                cp_c1 = pltpu.async_copy(
                    col_hbm.at[w, pl.ds(n_half, n_half)], col_v, semi)
                cp_r1 = pltpu.async_copy(
                    row_hbm.at[w, pl.ds(n_half, n_half)], row_v, semi)
                cp_c1.wait()
                cp_r1.wait()

            for b in range(_G):
                pltpu.async_copy(x_hbm.at[col_v.at[b]], buf0.at[b], sem0)

            @pl.loop(0, n_half, step=_G)
            def _grp(k0):
                for b in range(_G):
                    k = k0 + b
                    pltpu.make_async_copy(
                        x_hbm.at[col_v.at[k]], buf0.at[b], sem0).wait()
                    pltpu.sync_copy(
                        buf0.at[b], acc.at[row_v.at[k]], add=True)

                    @pl.when(k + _G < n_half)
                    def _issue():
                        pltpu.async_copy(
                            x_hbm.at[col_v.at[k + _G]], buf0.at[b], sem0)

        plsc.subcore_barrier()
        pltpu.sync_copy(acc.at[pl.ds(s * rpt, rpt)],
                        out_hbm.at[c, pl.ds(s * rpt, rpt)])

    return agg, n_chunks, e_pad, n_acc


def _tc_linear_body(x_ref, p0_ref, p1_ref, w_ref, b_ref, o_ref):
    s = x_ref[...] + p0_ref[...] + p1_ref[...]
    h = lax.dot_general(s, w_ref[...], (((1,), (1,)), ((), ())),
                        preferred_element_type=jnp.float32)
    h = h + b_ref[...]
    o_ref[...] = h * jax.nn.sigmoid(h)


def kernel(x, edge_index, edge_attr, W, b):
    N, D = x.shape
    E = edge_index.shape[1]
    NW = _NC * _NS
    ei = edge_index.astype(jnp.int32)
    row, col = ei[0], ei[1]

    agg_fn, n_chunks, e_pad, n_acc = _make_sc_agg(N, D, E)
    pad = e_pad - E
    n_dummy = n_acc - N
    row_p = jnp.concatenate(
        [row, N + (jnp.arange(pad, dtype=jnp.int32) % n_dummy)])
    row_p = row_p.reshape(NW, n_chunks, _CHUNK)
    col_p = jnp.concatenate(
        [col, jnp.arange(pad, dtype=jnp.int32) % N])
    col_p = col_p.reshape(NW, n_chunks, _CHUNK)
    zeros = jnp.zeros((n_acc // _NS, D), jnp.float32)

    parts = agg_fn(x, row_p, col_p, zeros)
    p0 = parts[0, :N]
    p1 = parts[1, :N]

    RB = 1000
    return pl.pallas_call(
        _tc_linear_body,
        grid=(N // RB,),
        in_specs=[
            pl.BlockSpec((RB, D), lambda i: (i, 0)),
            pl.BlockSpec((RB, D), lambda i: (i, 0)),
            pl.BlockSpec((RB, D), lambda i: (i, 0)),
            pl.BlockSpec((D, D), lambda i: (0, 0)),
            pl.BlockSpec((1, D), lambda i: (0, 0)),
        ],
        out_specs=pl.BlockSpec((RB, D), lambda i: (i, 0)),
        out_shape=jax.ShapeDtypeStruct((N, D), jnp.float32),
    )(x, p0, p1, W, b.reshape(1, D))
